# baseline (speedup 1.0000x reference)
import functools

import jax
import jax.numpy as jnp
from jax import lax
from jax.experimental import pallas as pl
from jax.experimental.pallas import tpu as pltpu
from jax.experimental.pallas import tpu_sc as plsc

_N = 4096 * 256
_NC, _NS, _L = 2, 16, 16
_NW = _NC * _NS
_CHUNK = _N // _NW
_UNROLL = 8


@functools.partial(
    pl.kernel,
    out_type=jax.ShapeDtypeStruct((_N,), jnp.int32),
    mesh=plsc.VectorSubcoreMesh(core_axis_name="c", subcore_axis_name="s"),
    scratch_types=[pltpu.VMEM((_CHUNK,), jnp.int32)],
)
def _remap(sp_hbm, out_hbm, buf):
    wid = lax.axis_index("s") * _NC + lax.axis_index("c")
    base = wid * _CHUNK
    pltpu.sync_copy(sp_hbm.at[pl.ds(base, _CHUNK)], buf)

    neg1 = jnp.full((_L,), -1, jnp.int32)

    def step(i, carry):
        off0 = i * (_L * _UNROLL)
        for u in range(_UNROLL):
            off = off0 + u * _L
            v = buf[pl.ds(off, _L)]
            w = v - 1
            ok = w.astype(jnp.uint32) < jnp.uint32(8)
            buf[pl.ds(off, _L)] = jnp.where(ok, w, neg1)
        return carry

    lax.fori_loop(0, _CHUNK // (_L * _UNROLL), step, 0)
    pltpu.sync_copy(buf, out_hbm.at[pl.ds(base, _CHUNK)])


def kernel(species, coordinates):
    flat = species.reshape(-1)
    out = _remap(flat)
    return out.reshape(species.shape), coordinates


# speedup vs baseline: 1.2510x; 1.2510x over previous
import functools

import jax
import jax.numpy as jnp
from jax import lax
from jax.experimental import pallas as pl
from jax.experimental.pallas import tpu as pltpu
from jax.experimental.pallas import tpu_sc as plsc

_R, _C = 4096, 256
_NC, _NS, _L = 2, 16, 16
_NW = _NC * _NS
_RW = _R // _NW


@functools.partial(
    pl.kernel,
    out_type=jax.ShapeDtypeStruct((_R, _C), jnp.int32),
    mesh=plsc.VectorSubcoreMesh(core_axis_name="c", subcore_axis_name="s"),
    scratch_types=[pltpu.VMEM((_RW, _C), jnp.int32)],
)
def _remap(sp_hbm, out_hbm, buf):
    wid = lax.axis_index("s") * _NC + lax.axis_index("c")
    r0 = wid * _RW
    pltpu.sync_copy(sp_hbm.at[pl.ds(r0, _RW), :], buf)

    neg1 = jnp.full((_L,), -1, jnp.int32)

    def step(r, carry):
        for u in range(_C // _L):
            v = buf[r, pl.ds(u * _L, _L)]
            w = v - 1
            ok = w.astype(jnp.uint32) < jnp.uint32(8)
            buf[r, pl.ds(u * _L, _L)] = jnp.where(ok, w, neg1)
        return carry

    lax.fori_loop(0, _RW, step, 0)
    pltpu.sync_copy(buf, out_hbm.at[pl.ds(r0, _RW), :])


def kernel(species, coordinates):
    return _remap(species), coordinates
